# baseline (speedup 1.0000x reference)
import functools

import jax
import jax.numpy as jnp
from jax import lax
from jax.experimental import pallas as pl
from jax.experimental.pallas import tpu as pltpu
from jax.experimental.pallas import tpu_sc as plsc

SHIFT = 18
NBUCKET = 16384
CH = 3
CHUNK_ROWS = 8
NVEC = CHUNK_ROWS * 512 // 16
NCHUNK = 64 // CHUNK_ROWS
HISTW = CH * 2 * NBUCKET
HROWS = HISTW // 128

_MESH = plsc.VectorSubcoreMesh(core_axis_name="c", subcore_axis_name="s")
_PARAMS = pltpu.CompilerParams(needs_layout_passes=False,
                               use_tc_tiling_on_sc=True)


def _worker_id():
    return lax.axis_index("s") * 2 + lax.axis_index("c")


def _chunk_slices(wid):
    out = []
    for ch in range(CH):
        for k in range(NCHUNK):
            out.append((ch, wid // 8, (wid % 8) * 64 + k * CHUNK_ROWS))
    return out


def _stage(logits_hbm, target_hbm, lbuf, tbuf, lsems, tsems, slices, j, slot):
    ch, b, r = slices[j]
    cl = pltpu.async_copy(logits_hbm.at[b, ch, pl.ds(r, CHUNK_ROWS), :],
                          lbuf.at[slot], lsems[slot])
    ct = pltpu.async_copy(target_hbm.at[b, ch, pl.ds(r, CHUNK_ROWS), :],
                          tbuf.at[slot], tsems[slot])
    return cl, ct


@functools.partial(
    pl.kernel,
    out_type=(
        jax.ShapeDtypeStruct((32, HROWS, 128), jnp.int32),
        jax.ShapeDtypeStruct((32, 8, 128), jnp.int32),
    ),
    mesh=_MESH,
    compiler_params=_PARAMS,
    scratch_types=[
        pltpu.VMEM((2, CHUNK_ROWS, 512), jnp.float32),
        pltpu.VMEM((2, CHUNK_ROWS, 512), jnp.int32),
        pltpu.VMEM((HROWS, 128), jnp.int32),
        pltpu.VMEM((8, 128), jnp.int32),
        pltpu.SemaphoreType.DMA,
        pltpu.SemaphoreType.DMA,
        pltpu.SemaphoreType.DMA,
        pltpu.SemaphoreType.DMA,
    ],
)
def _hist_kernel(logits_hbm, target_hbm, hist_hbm, pos_hbm,
                 lbuf, tbuf, histv, posv, lsem0, lsem1, tsem0, tsem1):
    lsems, tsems = (lsem0, lsem1), (tsem0, tsem1)
    wid = _worker_id()
    zeros16 = jnp.zeros((16,), jnp.int32)
    ones16 = jnp.ones((16,), jnp.int32)
    slices = _chunk_slices(wid)

    pend = _stage(logits_hbm, target_hbm, lbuf, tbuf, lsems, tsems,
                  slices, 0, 0)

    def _zero(i, _):
        histv[i // 8, pl.ds((i % 8) * 16, 16)] = zeros16
        return 0

    lax.fori_loop(0, HISTW // 16, _zero, 0, unroll=8)

    def _zerop(i, _):
        posv[i // 8, pl.ds((i % 8) * 16, 16)] = zeros16
        return 0

    lax.fori_loop(0, 64, _zerop, 0, unroll=8)

    paccs = {}
    for j in range(len(slices)):
        ch, _, _ = slices[j]
        slot = j & 1
        cl, ct = pend
        if j + 1 < len(slices):
            nxt = _stage(logits_hbm, target_hbm, lbuf, tbuf, lsems, tsems,
                         slices, j + 1, slot ^ 1)
        cl.wait()
        ct.wait()
        if j + 1 < len(slices):
            pend = nxt
        base = 2 * ch * NBUCKET

        def _vec(i, pacc, slot=slot, base=base):
            row = i // 32
            col = (i % 32) * 16
            l = lbuf[slot, row, pl.ds(col, 16)]
            t = tbuf[slot, row, pl.ds(col, 16)]
            tf = t.astype(jnp.float32)
            e = 1.0 - l * (2.0 * tf - 1.0)
            m = e > 0.0
            b = lax.shift_right_logical(
                lax.bitcast_convert_type(e, jnp.int32), SHIFT)
            idx = t * NBUCKET + b + base
            plsc.addupdate_scatter(histv, [lax.shift_right_logical(idx, 7),
                                           lax.bitwise_and(idx, 127)],
                                   ones16, mask=m)
            return pacc + t

        paccs[ch] = lax.fori_loop(0, NVEC, _vec,
                                  paccs.get(ch, zeros16), unroll=4)

    for ch in range(CH):
        posv[ch, pl.ds(0, 16)] = paccs[ch]

    pltpu.sync_copy(histv, hist_hbm.at[wid])
    pltpu.sync_copy(posv, pos_hbm.at[wid])


def _scan_body(hist_ref, pos_ref, m_ref, g_ref):
    h = hist_ref[...].reshape(32, CH * 2, NBUCKET)
    hf = jnp.sum(h, axis=0).astype(jnp.float32)
    nrow = NBUCKET // 128
    x = hf.reshape(CH * 2 * nrow, 128)
    io = lax.broadcasted_iota(jnp.int32, (128, 128), 0)
    jo = lax.broadcasted_iota(jnp.int32, (128, 128), 1)
    tri = (io <= jo).astype(jnp.float32)
    tri_s = (io < jo).astype(jnp.float32)
    inc = lax.dot(x, tri, precision=lax.Precision.HIGHEST,
                  preferred_element_type=jnp.float32)
    rows = inc[:, 127].reshape(CH * 2, nrow)
    off = lax.dot(rows, tri_s[:nrow, :nrow], precision=lax.Precision.HIGHEST,
                  preferred_element_type=jnp.float32)
    cum = (inc.reshape(CH * 2, nrow, 128) + off[:, :, None]).reshape(
        CH * 2, NBUCKET)
    total = cum[:, NBUCKET - 1 :]
    m_ref[...] = (total - cum + 0.5 * hf).reshape(HROWS, 128)
    g = jnp.sum(pos_ref[:, 0:CH, 0:16], axis=(0, 2)).astype(jnp.float32)
    g8 = jnp.concatenate([g, jnp.zeros((8 - CH,), jnp.float32)])
    g_ref[...] = jnp.broadcast_to(g8[:, None], (8, 128))


def _scan_tables(hist32, pos32):
    return pl.pallas_call(
        _scan_body,
        out_shape=(
            jax.ShapeDtypeStruct((HROWS, 128), jnp.float32),
            jax.ShapeDtypeStruct((8, 128), jnp.float32),
        ),
        in_specs=[
            pl.BlockSpec(memory_space=pltpu.VMEM),
            pl.BlockSpec(memory_space=pltpu.VMEM),
        ],
        out_specs=(
            pl.BlockSpec(memory_space=pltpu.VMEM),
            pl.BlockSpec(memory_space=pltpu.VMEM),
        ),
    )(hist32, pos32)


@functools.partial(
    pl.kernel,
    out_type=jax.ShapeDtypeStruct((32, 8, 128), jnp.float32),
    mesh=_MESH,
    compiler_params=_PARAMS,
    scratch_types=[
        pltpu.VMEM((2, CHUNK_ROWS, 512), jnp.float32),
        pltpu.VMEM((2, CHUNK_ROWS, 512), jnp.int32),
        pltpu.VMEM((HROWS, 128), jnp.float32),
        pltpu.VMEM((8, 128), jnp.float32),
        pltpu.VMEM((8, 128), jnp.float32),
        pltpu.SemaphoreType.DMA,
        pltpu.SemaphoreType.DMA,
        pltpu.SemaphoreType.DMA,
        pltpu.SemaphoreType.DMA,
    ],
)
def _eval_kernel(logits_hbm, target_hbm, m_hbm, g_hbm, out_hbm,
                 lbuf, tbuf, mtab, gbuf, accv, lsem0, lsem1, tsem0, tsem1):
    lsems, tsems = (lsem0, lsem1), (tsem0, tsem1)
    wid = _worker_id()
    slices = _chunk_slices(wid)

    pend = _stage(logits_hbm, target_hbm, lbuf, tbuf, lsems, tsems,
                  slices, 0, 0)
    pltpu.sync_copy(m_hbm, mtab)
    pltpu.sync_copy(g_hbm, gbuf)

    zeros16 = jnp.zeros((16,), jnp.float32)

    def _zeroa(i, _):
        accv[i // 8, pl.ds((i % 8) * 16, 16)] = zeros16
        return 0

    lax.fori_loop(0, 64, _zeroa, 0, unroll=8)

    accs = {}
    for j in range(len(slices)):
        ch, _, _ = slices[j]
        slot = j & 1
        cl, ct = pend
        if j + 1 < len(slices):
            nxt = _stage(logits_hbm, target_hbm, lbuf, tbuf, lsems, tsems,
                         slices, j + 1, slot ^ 1)
        cl.wait()
        ct.wait()
        if j + 1 < len(slices):
            pend = nxt
        base = 2 * ch * NBUCKET
        gv = gbuf[ch, pl.ds(0, 16)]

        def _vec(i, acc, slot=slot, base=base, gv=gv):
            row = i // 32
            col = (i % 32) * 16
            l = lbuf[slot, row, pl.ds(col, 16)]
            t = tbuf[slot, row, pl.ds(col, 16)]
            tf = t.astype(jnp.float32)
            e = 1.0 - l * (2.0 * tf - 1.0)
            m = e > 0.0
            b = lax.shift_right_logical(
                lax.bitcast_convert_type(e, jnp.int32), SHIFT)
            bn = b + base
            bp = bn + NBUCKET
            mn = plsc.load_gather(mtab, [lax.shift_right_logical(bn, 7),
                                         lax.bitwise_and(bn, 127)], mask=m)
            mp = plsc.load_gather(mtab, [lax.shift_right_logical(bp, 7),
                                         lax.bitwise_and(bp, 127)],
                                  mask=m & (t == 0))
            n = mn - 0.5 * (1.0 - tf)
            d1 = gv + n
            d2 = d1 + 1.0
            num = jnp.where(t == 0, gv - mp, d2)
            contrib = jnp.where(m, e * num / (d1 * d2), 0.0)
            return acc + contrib

        accs[ch] = lax.fori_loop(0, NVEC, _vec,
                                 accs.get(ch, zeros16), unroll=4)

    for ch in range(CH):
        accv[ch, pl.ds(0, 16)] = accs[ch]

    pltpu.sync_copy(accv, out_hbm.at[wid])


def _finish_body(part_ref, out_ref):
    s = jnp.sum(part_ref[:, 0:CH, 0:16])
    out_ref[0, 0] = s / (CH + 1e-06)


def _finish(partials):
    return pl.pallas_call(
        _finish_body,
        out_shape=jax.ShapeDtypeStruct((1, 1), jnp.float32),
        in_specs=[pl.BlockSpec(memory_space=pltpu.VMEM)],
        out_specs=pl.BlockSpec(memory_space=pltpu.SMEM),
    )(partials)


def kernel(logits, target):
    hist32, pos32 = _hist_kernel(logits, target)
    mtab, g = _scan_tables(hist32, pos32)
    partials = _eval_kernel(logits, target, mtab, g)
    out = _finish(partials)
    return out.reshape(())


# speedup vs baseline: 1.7545x; 1.7545x over previous
import functools

import jax
import jax.numpy as jnp
from jax import lax
from jax.experimental import pallas as pl
from jax.experimental.pallas import tpu as pltpu
from jax.experimental.pallas import tpu_sc as plsc

SHIFT = 19
NBUCKET = 8192
CH = 3
CHUNK_ROWS = 16
NVEC = CHUNK_ROWS * 512 // 16
NCHUNK = 64 // CHUNK_ROWS
HISTW = CH * 2 * NBUCKET
HROWS = HISTW // 128

_MESH = plsc.VectorSubcoreMesh(core_axis_name="c", subcore_axis_name="s")
_PARAMS = pltpu.CompilerParams(needs_layout_passes=False,
                               use_tc_tiling_on_sc=True)


def _worker_id():
    return lax.axis_index("s") * 2 + lax.axis_index("c")


def _chunk_slices(wid):
    out = []
    for ch in range(CH):
        for k in range(NCHUNK):
            out.append((ch, wid // 8, (wid % 8) * 64 + k * CHUNK_ROWS))
    return out


def _stage(logits_hbm, target_hbm, lbuf, tbuf, lsems, tsems, slices, j, slot):
    ch, b, r = slices[j]
    cl = pltpu.async_copy(logits_hbm.at[b, ch, pl.ds(r, CHUNK_ROWS), :],
                          lbuf.at[slot], lsems[slot])
    ct = pltpu.async_copy(target_hbm.at[b, ch, pl.ds(r, CHUNK_ROWS), :],
                          tbuf.at[slot], tsems[slot])
    return cl, ct


@functools.partial(
    pl.kernel,
    out_type=(
        jax.ShapeDtypeStruct((32, HROWS, 128), jnp.int32),
        jax.ShapeDtypeStruct((32, 8, 128), jnp.int32),
    ),
    mesh=_MESH,
    compiler_params=_PARAMS,
    scratch_types=[
        pltpu.VMEM((2, CHUNK_ROWS, 512), jnp.float32),
        pltpu.VMEM((2, CHUNK_ROWS, 512), jnp.int32),
        pltpu.VMEM((HROWS, 128), jnp.int32),
        pltpu.VMEM((8, 128), jnp.int32),
        pltpu.SemaphoreType.DMA,
        pltpu.SemaphoreType.DMA,
        pltpu.SemaphoreType.DMA,
        pltpu.SemaphoreType.DMA,
    ],
)
def _hist_kernel(logits_hbm, target_hbm, hist_hbm, pos_hbm,
                 lbuf, tbuf, histv, posv, lsem0, lsem1, tsem0, tsem1):
    lsems, tsems = (lsem0, lsem1), (tsem0, tsem1)
    wid = _worker_id()
    zeros16 = jnp.zeros((16,), jnp.int32)
    ones16 = jnp.ones((16,), jnp.int32)
    slices = _chunk_slices(wid)

    pend = _stage(logits_hbm, target_hbm, lbuf, tbuf, lsems, tsems,
                  slices, 0, 0)

    def _zero(i, _):
        histv[i // 8, pl.ds((i % 8) * 16, 16)] = zeros16
        return 0

    lax.fori_loop(0, HISTW // 16, _zero, 0, unroll=8)

    def _zerop(i, _):
        posv[i // 8, pl.ds((i % 8) * 16, 16)] = zeros16
        return 0

    lax.fori_loop(0, 64, _zerop, 0, unroll=8)

    paccs = {}
    for j in range(len(slices)):
        ch, _, _ = slices[j]
        slot = j & 1
        cl, ct = pend
        if j + 1 < len(slices):
            nxt = _stage(logits_hbm, target_hbm, lbuf, tbuf, lsems, tsems,
                         slices, j + 1, slot ^ 1)
        cl.wait()
        ct.wait()
        if j + 1 < len(slices):
            pend = nxt
        base = 2 * ch * NBUCKET

        def _vec(g, pacc, slot=slot, base=base):
            pend_scatter = []
            for u in range(4):
                i = g * 4 + u
                row = i // 32
                col = (i % 32) * 16
                l = lbuf[slot, row, pl.ds(col, 16)]
                t = tbuf[slot, row, pl.ds(col, 16)]
                tf = t.astype(jnp.float32)
                e = 1.0 - l * (2.0 * tf - 1.0)
                m = e > 0.0
                b = lax.shift_right_logical(
                    lax.bitcast_convert_type(e, jnp.int32), SHIFT)
                idx = t * NBUCKET + b + base
                pend_scatter.append((idx, m))
                pacc = pacc + t
            for idx, m in pend_scatter:
                plsc.addupdate_scatter(histv,
                                       [lax.shift_right_logical(idx, 7),
                                        lax.bitwise_and(idx, 127)],
                                       ones16, mask=m)
            return pacc

        paccs[ch] = lax.fori_loop(0, NVEC // 4, _vec,
                                  paccs.get(ch, zeros16))

    for ch in range(CH):
        posv[ch, pl.ds(0, 16)] = paccs[ch]

    pltpu.sync_copy(histv, hist_hbm.at[wid])
    pltpu.sync_copy(posv, pos_hbm.at[wid])


def _scan_body(hist_ref, pos_ref, m_ref, g_ref):
    h = hist_ref[...].reshape(32, CH * 2, NBUCKET)
    hf = jnp.sum(h, axis=0).astype(jnp.float32)
    nrow = NBUCKET // 128
    x = hf.reshape(CH * 2 * nrow, 128)
    io = lax.broadcasted_iota(jnp.int32, (128, 128), 0)
    jo = lax.broadcasted_iota(jnp.int32, (128, 128), 1)
    tri = (io <= jo).astype(jnp.float32)
    tri_s = (io < jo).astype(jnp.float32)
    inc = lax.dot(x, tri, precision=lax.Precision.HIGHEST,
                  preferred_element_type=jnp.float32)
    rows = inc[:, 127].reshape(CH * 2, nrow)
    off = lax.dot(rows, tri_s[:nrow, :nrow], precision=lax.Precision.HIGHEST,
                  preferred_element_type=jnp.float32)
    cum = (inc.reshape(CH * 2, nrow, 128) + off[:, :, None]).reshape(
        CH * 2, NBUCKET)
    total = cum[:, NBUCKET - 1 :]
    m_ref[...] = (total - cum + 0.5 * hf).reshape(HROWS, 128)
    g = jnp.sum(pos_ref[:, 0:CH, 0:16], axis=(0, 2)).astype(jnp.float32)
    g8 = jnp.concatenate([g, jnp.zeros((8 - CH,), jnp.float32)])
    g_ref[...] = jnp.broadcast_to(g8[:, None], (8, 128))


def _scan_tables(hist32, pos32):
    return pl.pallas_call(
        _scan_body,
        out_shape=(
            jax.ShapeDtypeStruct((HROWS, 128), jnp.float32),
            jax.ShapeDtypeStruct((8, 128), jnp.float32),
        ),
        in_specs=[
            pl.BlockSpec(memory_space=pltpu.VMEM),
            pl.BlockSpec(memory_space=pltpu.VMEM),
        ],
        out_specs=(
            pl.BlockSpec(memory_space=pltpu.VMEM),
            pl.BlockSpec(memory_space=pltpu.VMEM),
        ),
    )(hist32, pos32)


@functools.partial(
    pl.kernel,
    out_type=jax.ShapeDtypeStruct((32, 8, 128), jnp.float32),
    mesh=_MESH,
    compiler_params=_PARAMS,
    scratch_types=[
        pltpu.VMEM((2, CHUNK_ROWS, 512), jnp.float32),
        pltpu.VMEM((2, CHUNK_ROWS, 512), jnp.int32),
        pltpu.VMEM((HROWS, 128), jnp.float32),
        pltpu.VMEM((8, 128), jnp.float32),
        pltpu.VMEM((8, 128), jnp.float32),
        pltpu.SemaphoreType.DMA,
        pltpu.SemaphoreType.DMA,
        pltpu.SemaphoreType.DMA,
        pltpu.SemaphoreType.DMA,
    ],
)
def _eval_kernel(logits_hbm, target_hbm, m_hbm, g_hbm, out_hbm,
                 lbuf, tbuf, mtab, gbuf, accv, lsem0, lsem1, tsem0, tsem1):
    lsems, tsems = (lsem0, lsem1), (tsem0, tsem1)
    wid = _worker_id()
    slices = _chunk_slices(wid)

    pend = _stage(logits_hbm, target_hbm, lbuf, tbuf, lsems, tsems,
                  slices, 0, 0)
    pltpu.sync_copy(m_hbm, mtab)
    pltpu.sync_copy(g_hbm, gbuf)

    zeros16 = jnp.zeros((16,), jnp.float32)

    def _zeroa(i, _):
        accv[i // 8, pl.ds((i % 8) * 16, 16)] = zeros16
        return 0

    lax.fori_loop(0, 64, _zeroa, 0, unroll=8)

    accs = {}
    for j in range(len(slices)):
        ch, _, _ = slices[j]
        slot = j & 1
        cl, ct = pend
        if j + 1 < len(slices):
            nxt = _stage(logits_hbm, target_hbm, lbuf, tbuf, lsems, tsems,
                         slices, j + 1, slot ^ 1)
        cl.wait()
        ct.wait()
        if j + 1 < len(slices):
            pend = nxt
        base = 2 * ch * NBUCKET
        gv = gbuf[ch, pl.ds(0, 16)]

        def _vec(i, acc, slot=slot, base=base, gv=gv):
            row = i // 32
            col = (i % 32) * 16
            l = lbuf[slot, row, pl.ds(col, 16)]
            t = tbuf[slot, row, pl.ds(col, 16)]
            tf = t.astype(jnp.float32)
            e = 1.0 - l * (2.0 * tf - 1.0)
            m = e > 0.0
            b = lax.shift_right_logical(
                lax.bitcast_convert_type(e, jnp.int32), SHIFT)
            bn = b + base
            bp = bn + NBUCKET
            mn = plsc.load_gather(mtab, [lax.shift_right_logical(bn, 7),
                                         lax.bitwise_and(bn, 127)], mask=m)
            mp = plsc.load_gather(mtab, [lax.shift_right_logical(bp, 7),
                                         lax.bitwise_and(bp, 127)],
                                  mask=m & (t == 0))
            n = mn - 0.5 * (1.0 - tf)
            d1 = gv + n
            d2 = d1 + 1.0
            num = jnp.where(t == 0, gv - mp, d2)
            contrib = jnp.where(m, e * num / (d1 * d2), 0.0)
            return acc + contrib

        accs[ch] = lax.fori_loop(0, NVEC, _vec,
                                 accs.get(ch, zeros16), unroll=4)

    for ch in range(CH):
        accv[ch, pl.ds(0, 16)] = accs[ch]

    pltpu.sync_copy(accv, out_hbm.at[wid])


def _finish_body(part_ref, out_ref):
    s = jnp.sum(part_ref[:, 0:CH, 0:16])
    out_ref[0, 0] = s / (CH + 1e-06)


def _finish(partials):
    return pl.pallas_call(
        _finish_body,
        out_shape=jax.ShapeDtypeStruct((1, 1), jnp.float32),
        in_specs=[pl.BlockSpec(memory_space=pltpu.VMEM)],
        out_specs=pl.BlockSpec(memory_space=pltpu.SMEM),
    )(partials)


def kernel(logits, target):
    hist32, pos32 = _hist_kernel(logits, target)
    mtab, g = _scan_tables(hist32, pos32)
    partials = _eval_kernel(logits, target, mtab, g)
    out = _finish(partials)
    return out.reshape(())


# speedup vs baseline: 1.8606x; 1.0605x over previous
import functools

import jax
import jax.numpy as jnp
from jax import lax
from jax.experimental import pallas as pl
from jax.experimental.pallas import tpu as pltpu
from jax.experimental.pallas import tpu_sc as plsc

SHIFT = 19
NBUCKET = 8192
CH = 3
CHUNK_ROWS = 16
NVEC = CHUNK_ROWS * 512 // 16
NCHUNK = 64 // CHUNK_ROWS
HISTW = CH * 2 * NBUCKET
HROWS = HISTW // 128

_MESH = plsc.VectorSubcoreMesh(core_axis_name="c", subcore_axis_name="s")
_PARAMS = pltpu.CompilerParams(needs_layout_passes=False,
                               use_tc_tiling_on_sc=True)


def _worker_id():
    return lax.axis_index("s") * 2 + lax.axis_index("c")


def _chunk_slices(wid):
    out = []
    for ch in range(CH):
        for k in range(NCHUNK):
            out.append((ch, wid // 8, (wid % 8) * 64 + k * CHUNK_ROWS))
    return out


def _stage(logits_hbm, target_hbm, lbuf, tbuf, lsems, tsems, slices, j, slot):
    ch, b, r = slices[j]
    cl = pltpu.async_copy(logits_hbm.at[b, ch, pl.ds(r, CHUNK_ROWS), :],
                          lbuf.at[slot], lsems[slot])
    ct = pltpu.async_copy(target_hbm.at[b, ch, pl.ds(r, CHUNK_ROWS), :],
                          tbuf.at[slot], tsems[slot])
    return cl, ct


@functools.partial(
    pl.kernel,
    out_type=(
        jax.ShapeDtypeStruct((32, HROWS, 128), jnp.int32),
        jax.ShapeDtypeStruct((32, 8, 128), jnp.int32),
    ),
    mesh=_MESH,
    compiler_params=_PARAMS,
    scratch_types=[
        pltpu.VMEM((2, CHUNK_ROWS, 512), jnp.float32),
        pltpu.VMEM((2, CHUNK_ROWS, 512), jnp.int32),
        pltpu.VMEM((HROWS, 128), jnp.int32),
        pltpu.VMEM((8, 128), jnp.int32),
        pltpu.SemaphoreType.DMA,
        pltpu.SemaphoreType.DMA,
        pltpu.SemaphoreType.DMA,
        pltpu.SemaphoreType.DMA,
    ],
)
def _hist_kernel(logits_hbm, target_hbm, hist_hbm, pos_hbm,
                 lbuf, tbuf, histv, posv, lsem0, lsem1, tsem0, tsem1):
    lsems, tsems = (lsem0, lsem1), (tsem0, tsem1)
    wid = _worker_id()
    zeros16 = jnp.zeros((16,), jnp.int32)
    ones16 = jnp.ones((16,), jnp.int32)
    slices = _chunk_slices(wid)

    pend = _stage(logits_hbm, target_hbm, lbuf, tbuf, lsems, tsems,
                  slices, 0, 0)

    def _zero(i, _):
        histv[i // 8, pl.ds((i % 8) * 16, 16)] = zeros16
        return 0

    lax.fori_loop(0, HISTW // 16, _zero, 0, unroll=8)

    def _zerop(i, _):
        posv[i // 8, pl.ds((i % 8) * 16, 16)] = zeros16
        return 0

    lax.fori_loop(0, 64, _zerop, 0, unroll=8)

    paccs = {}
    for j in range(len(slices)):
        ch, _, _ = slices[j]
        slot = j & 1
        cl, ct = pend
        if j + 1 < len(slices):
            nxt = _stage(logits_hbm, target_hbm, lbuf, tbuf, lsems, tsems,
                         slices, j + 1, slot ^ 1)
        cl.wait()
        ct.wait()
        if j + 1 < len(slices):
            pend = nxt
        base = 2 * ch * NBUCKET

        def _vec(g, pacc, slot=slot, base=base):
            pend_scatter = []
            for u in range(4):
                i = g * 4 + u
                row = i // 32
                col = (i % 32) * 16
                l = lbuf[slot, row, pl.ds(col, 16)]
                t = tbuf[slot, row, pl.ds(col, 16)]
                neg = t == 0
                e = jnp.where(neg, 1.0 + l, 1.0 - l)
                m = e > 0.0
                b = lax.shift_right_logical(
                    lax.bitcast_convert_type(e, jnp.int32), SHIFT)
                idx = b + jnp.where(neg, base, base + NBUCKET)
                pend_scatter.append((idx, m))
                pacc = pacc + t
            for idx, m in pend_scatter:
                plsc.addupdate_scatter(histv,
                                       [lax.shift_right_logical(idx, 7),
                                        lax.bitwise_and(idx, 127)],
                                       ones16, mask=m)
            return pacc

        paccs[ch] = lax.fori_loop(0, NVEC // 4, _vec,
                                  paccs.get(ch, zeros16))

    for ch in range(CH):
        posv[ch, pl.ds(0, 16)] = paccs[ch]

    pltpu.sync_copy(histv, hist_hbm.at[wid])
    pltpu.sync_copy(posv, pos_hbm.at[wid])


def _scan_body(hist_ref, pos_ref, m_ref, g_ref):
    h = hist_ref[...].reshape(32, CH * 2, NBUCKET)
    hf = jnp.sum(h, axis=0).astype(jnp.float32)
    nrow = NBUCKET // 128
    x = hf.reshape(CH * 2 * nrow, 128)
    io = lax.broadcasted_iota(jnp.int32, (128, 128), 0)
    jo = lax.broadcasted_iota(jnp.int32, (128, 128), 1)
    tri = (io <= jo).astype(jnp.float32)
    tri_s = (io < jo).astype(jnp.float32)
    inc = lax.dot(x, tri, precision=lax.Precision.HIGHEST,
                  preferred_element_type=jnp.float32)
    rows = inc[:, 127].reshape(CH * 2, nrow)
    off = lax.dot(rows, tri_s[:nrow, :nrow], precision=lax.Precision.HIGHEST,
                  preferred_element_type=jnp.float32)
    cum = (inc.reshape(CH * 2, nrow, 128) + off[:, :, None]).reshape(
        CH * 2, NBUCKET)
    total = cum[:, NBUCKET - 1 :]
    m_ref[...] = (total - cum + 0.5 * hf).reshape(HROWS, 128)
    g = jnp.sum(pos_ref[:, 0:CH, 0:16], axis=(0, 2)).astype(jnp.float32)
    g8 = jnp.concatenate([g, jnp.zeros((8 - CH,), jnp.float32)])
    g_ref[...] = jnp.broadcast_to(g8[:, None], (8, 128))


def _scan_tables(hist32, pos32):
    return pl.pallas_call(
        _scan_body,
        out_shape=(
            jax.ShapeDtypeStruct((HROWS, 128), jnp.float32),
            jax.ShapeDtypeStruct((8, 128), jnp.float32),
        ),
        in_specs=[
            pl.BlockSpec(memory_space=pltpu.VMEM),
            pl.BlockSpec(memory_space=pltpu.VMEM),
        ],
        out_specs=(
            pl.BlockSpec(memory_space=pltpu.VMEM),
            pl.BlockSpec(memory_space=pltpu.VMEM),
        ),
    )(hist32, pos32)


@functools.partial(
    pl.kernel,
    out_type=jax.ShapeDtypeStruct((32, 8, 128), jnp.float32),
    mesh=_MESH,
    compiler_params=_PARAMS,
    scratch_types=[
        pltpu.VMEM((2, CHUNK_ROWS, 512), jnp.float32),
        pltpu.VMEM((2, CHUNK_ROWS, 512), jnp.int32),
        pltpu.VMEM((HROWS, 128), jnp.float32),
        pltpu.VMEM((8, 128), jnp.float32),
        pltpu.VMEM((8, 128), jnp.float32),
        pltpu.SemaphoreType.DMA,
        pltpu.SemaphoreType.DMA,
        pltpu.SemaphoreType.DMA,
        pltpu.SemaphoreType.DMA,
    ],
)
def _eval_kernel(logits_hbm, target_hbm, m_hbm, g_hbm, out_hbm,
                 lbuf, tbuf, mtab, gbuf, accv, lsem0, lsem1, tsem0, tsem1):
    lsems, tsems = (lsem0, lsem1), (tsem0, tsem1)
    wid = _worker_id()
    slices = _chunk_slices(wid)

    pend = _stage(logits_hbm, target_hbm, lbuf, tbuf, lsems, tsems,
                  slices, 0, 0)
    pltpu.sync_copy(m_hbm, mtab)
    pltpu.sync_copy(g_hbm, gbuf)

    zeros16 = jnp.zeros((16,), jnp.float32)

    def _zeroa(i, _):
        accv[i // 8, pl.ds((i % 8) * 16, 16)] = zeros16
        return 0

    lax.fori_loop(0, 64, _zeroa, 0, unroll=8)

    accs = {}
    for j in range(len(slices)):
        ch, _, _ = slices[j]
        slot = j & 1
        cl, ct = pend
        if j + 1 < len(slices):
            nxt = _stage(logits_hbm, target_hbm, lbuf, tbuf, lsems, tsems,
                         slices, j + 1, slot ^ 1)
        cl.wait()
        ct.wait()
        if j + 1 < len(slices):
            pend = nxt
        base = 2 * ch * NBUCKET
        gv = gbuf[ch, pl.ds(0, 16)]

        gv05 = gv - 0.5

        def _vec(i, acc, slot=slot, base=base, gv=gv, gv05=gv05):
            row = i // 32
            col = (i % 32) * 16
            l = lbuf[slot, row, pl.ds(col, 16)]
            t = tbuf[slot, row, pl.ds(col, 16)]
            neg = t == 0
            e = jnp.where(neg, 1.0 + l, 1.0 - l)
            m = e > 0.0
            b = lax.shift_right_logical(
                lax.bitcast_convert_type(e, jnp.int32), SHIFT)
            bn = b + base
            bp = bn + NBUCKET
            mn = plsc.load_gather(mtab, [lax.shift_right_logical(bn, 7),
                                         lax.bitwise_and(bn, 127)], mask=m)
            mp = plsc.load_gather(mtab, [lax.shift_right_logical(bp, 7),
                                         lax.bitwise_and(bp, 127)],
                                  mask=m & neg)
            d1 = jnp.where(neg, gv05, gv) + mn
            d2 = d1 + 1.0
            num = jnp.where(neg, gv - mp, d2)
            contrib = jnp.where(m, e * num / (d1 * d2), 0.0)
            return acc + contrib

        accs[ch] = lax.fori_loop(0, NVEC, _vec,
                                 accs.get(ch, zeros16), unroll=4)

    for ch in range(CH):
        accv[ch, pl.ds(0, 16)] = accs[ch]

    pltpu.sync_copy(accv, out_hbm.at[wid])


def _finish_body(part_ref, out_ref):
    s = jnp.sum(part_ref[:, 0:CH, 0:16])
    out_ref[0, 0] = s / (CH + 1e-06)


def _finish(partials):
    return pl.pallas_call(
        _finish_body,
        out_shape=jax.ShapeDtypeStruct((1, 1), jnp.float32),
        in_specs=[pl.BlockSpec(memory_space=pltpu.VMEM)],
        out_specs=pl.BlockSpec(memory_space=pltpu.SMEM),
    )(partials)


def kernel(logits, target):
    hist32, pos32 = _hist_kernel(logits, target)
    mtab, g = _scan_tables(hist32, pos32)
    partials = _eval_kernel(logits, target, mtab, g)
    out = _finish(partials)
    return out.reshape(())
